# baseline (speedup 1.0000x reference)
import functools

import jax
import jax.numpy as jnp
from jax import lax
from jax.experimental import pallas as pl
from jax.experimental.pallas import tpu as pltpu
from jax.experimental.pallas import tpu_sc as plsc

_VOCAB = 1000000
_EMBED_DIM = 64
_BATCH = 1024
_CAP_LEN = 50

_NC = 2
_NS = 16
_NW = _NC * _NS

_B = _BATCH * _CAP_LEN
_B_PER_W = _B // _NW


def _make_gather():
    mesh = plsc.VectorSubcoreMesh(core_axis_name="c", subcore_axis_name="s")

    @functools.partial(
        pl.kernel,
        mesh=mesh,
        out_type=jax.ShapeDtypeStruct((_B, _EMBED_DIM), jnp.float32),
        scratch_types=[
            pltpu.VMEM((_B_PER_W,), jnp.int32),
            pltpu.VMEM((_B_PER_W, _EMBED_DIM), jnp.float32),
            pltpu.SemaphoreType.DMA,
        ],
        compiler_params=pltpu.CompilerParams(use_tc_tiling_on_sc=False),
    )
    def gather_k(table_hbm, idx_hbm, out_hbm, idx_v, rows_v, sem):
        wid = lax.axis_index("s") * _NC + lax.axis_index("c")
        base = wid * _B_PER_W
        pltpu.sync_copy(idx_hbm.at[pl.ds(base, _B_PER_W)], idx_v)
        pltpu.async_copy(table_hbm.at[idx_v], rows_v, sem).wait()
        pltpu.sync_copy(rows_v, out_hbm.at[pl.ds(base, _B_PER_W)])

    return gather_k


_gather = _make_gather()


def kernel(encoder_out, encoded_captions, caption_lengths, table):
    flat_idx = encoded_captions.reshape(_B)
    out = _gather(table, flat_idx)
    return out.reshape(_BATCH, _CAP_LEN, _EMBED_DIM)


# speedup vs baseline: 1.5984x; 1.5984x over previous
import functools

import jax
import jax.numpy as jnp
from jax import lax
from jax.experimental import pallas as pl
from jax.experimental.pallas import tpu as pltpu
from jax.experimental.pallas import tpu_sc as plsc

_VOCAB = 1000000
_EMBED_DIM = 64
_BATCH = 1024
_CAP_LEN = 50

_NC = 2
_NS = 16
_NW = _NC * _NS
_L = 16

_B = _BATCH * _CAP_LEN
_B_PER_W = _B // _NW
_CHUNK = 32
_N_CHUNKS = _B_PER_W // _CHUNK
_NTILES = _VOCAB // 8


def _make_gather():
    mesh = plsc.VectorSubcoreMesh(core_axis_name="c", subcore_axis_name="s")

    @functools.partial(
        pl.kernel,
        mesh=mesh,
        out_type=jax.ShapeDtypeStruct((_B, _EMBED_DIM), jnp.float32),
        scratch_types=[
            pltpu.VMEM((_B_PER_W,), jnp.int32),
            pltpu.VMEM((_CHUNK, 8, _EMBED_DIM), jnp.float32),
            pltpu.VMEM((_CHUNK, _EMBED_DIM), jnp.float32),
            pltpu.SemaphoreType.DMA,
            pltpu.SemaphoreType.DMA,
        ],
    )
    def gather_k(table_hbm, idx_hbm, out_hbm,
                 idx_v, tiles_v, rows_v, gsem, isem):
        wid = lax.axis_index("s") * _NC + lax.axis_index("c")
        base = wid * _B_PER_W

        pltpu.async_copy(
            idx_hbm.at[pl.ds(base, _B_PER_W)], idx_v, isem
        ).wait()

        def chunk_body(ch, _):
            off = ch * _CHUNK
            for g in range(_CHUNK // _L):
                tv = lax.shift_right_logical(idx_v[pl.ds(off + g * _L, _L)], 3)
                for k in range(_L):
                    pltpu.make_async_copy(
                        table_hbm.at[pl.ds(tv[k], 1)],
                        tiles_v.at[pl.ds(g * _L + k, 1)],
                        gsem,
                    ).start()

            pltpu.make_async_copy(
                table_hbm.at[pl.ds(0, _CHUNK)], tiles_v, gsem
            ).wait()

            for g in range(_CHUNK // _L):
                sv = idx_v[pl.ds(off + g * _L, _L)] & 7
                for k in range(_L):
                    j = g * _L + k
                    s = sv[k]
                    for c in range(_EMBED_DIM // _L):
                        rows_v[j, pl.ds(c * _L, _L)] = tiles_v[j, s, pl.ds(c * _L, _L)]

            pltpu.sync_copy(rows_v, out_hbm.at[pl.ds(base + off, _CHUNK)])
            return _

        lax.fori_loop(0, _N_CHUNKS, chunk_body, 0)

    return gather_k


_gather = _make_gather()


def kernel(encoder_out, encoded_captions, caption_lengths, table):
    flat_idx = encoded_captions.reshape(_B)
    table3 = table.reshape(_NTILES, 8, _EMBED_DIM)
    out = _gather(table3, flat_idx)
    return out.reshape(_BATCH, _CAP_LEN, _EMBED_DIM)
